# baseline (speedup 1.0000x reference)
import functools

import jax
import jax.numpy as jnp
from jax import lax
from jax.experimental import pallas as pl
from jax.experimental.pallas import tpu as pltpu
from jax.experimental.pallas import tpu_sc as plsc

BATCH = 4096
SEQ = 200
HIDDEN = 64
VOCAB = 1000000

_info = plsc.get_sparse_core_info()
NC, NS = _info.num_cores, _info.num_subcores
NW = NC * NS
B_PER_W = BATCH // NW
NBUF = 4
NOB = 2
GROUPS = SEQ // NBUF
L = 16


@functools.partial(
    pl.kernel,
    out_type=jax.ShapeDtypeStruct((SEQ, HIDDEN, BATCH), jnp.float32),
    mesh=plsc.VectorSubcoreMesh(core_axis_name="c", subcore_axis_name="s"),
    scratch_types=[
        pltpu.VMEM((SEQ, B_PER_W), jnp.int32),
        pltpu.VMEM((NBUF, B_PER_W), jnp.int32),
        pltpu.VMEM((NBUF, B_PER_W, 128), jnp.float32),
        pltpu.VMEM((NOB, HIDDEN, B_PER_W), jnp.float32),
        pltpu.SemaphoreType.DMA((NBUF,)),
        pltpu.SemaphoreType.DMA((NOB,)),
    ],
    compiler_params=pltpu.CompilerParams(needs_layout_passes=False),
)
def _gather_kernel(table_hbm, idx_hbm, out_hbm, idx_v, pv_v, pair_v, outb_v,
                   gsem, wsem):
    wid = lax.axis_index("s") * NC + lax.axis_index("c")
    pltpu.sync_copy(idx_hbm.at[wid], idx_v)

    def prep(s, b):
        for g in range(B_PER_W // L):
            v = idx_v[s, pl.ds(g * L, L)]
            pv_v[b, pl.ds(g * L, L)] = lax.shift_right_logical(v, 1)

    def gather(b):
        return pltpu.make_async_copy(
            table_hbm.at[pv_v.at[b]], pair_v.at[b], gsem.at[b])

    def writeback(s, ob):
        return pltpu.make_async_copy(
            outb_v.at[ob], out_hbm.at[s, :, pl.ds(wid * B_PER_W, B_PER_W)],
            wsem.at[ob])

    def transpose(s, b, ob):
        pair = pair_v.at[b]
        for g in range(B_PER_W // L):
            rows = lax.iota(jnp.int32, L) + (g * L)
            v = idx_v[s, pl.ds(g * L, L)]
            coloff = lax.shift_left(v & 1, 6)
            for h in range(HIDDEN):
                val = plsc.load_gather(pair, [rows, coloff + h])
                outb_v[ob, h, pl.ds(g * L, L)] = val

    for b in range(NBUF):
        prep(b, b)
        gather(b).start()

    def group(g, carry):
        for b in range(NBUF):
            s = g * NBUF + b
            ob = b & 1
            gather(b).wait()

            @pl.when(s >= NOB)
            def _():
                writeback(s - NOB, ob).wait()

            transpose(s, b, ob)
            writeback(s, ob).start()

            @pl.when(s + NBUF < SEQ)
            def _():
                prep(s + NBUF, b)
                gather(b).start()

        return carry

    lax.fori_loop(0, GROUPS, group, 0)
    writeback(SEQ - 2, 0).wait()
    writeback(SEQ - 1, 1).wait()


def kernel(inp_seq, inp_seq_len, embedding_table):
    del inp_seq_len
    table2 = embedding_table.reshape(VOCAB // 2, 128)
    idx_t = (
        inp_seq.astype(jnp.int32)
        .reshape(NW, B_PER_W, SEQ)
        .transpose(0, 2, 1)
    )
    out_t = _gather_kernel(table2, idx_t)
    return jnp.transpose(out_t, (2, 0, 1))


# speedup vs baseline: 1.9151x; 1.9151x over previous
import functools

import jax
import jax.numpy as jnp
from jax import lax
from jax.experimental import pallas as pl
from jax.experimental.pallas import tpu as pltpu
from jax.experimental.pallas import tpu_sc as plsc

BATCH = 4096
SEQ = 200
HIDDEN = 64
VOCAB = 1000000

_info = plsc.get_sparse_core_info()
NC, NS = _info.num_cores, _info.num_subcores
NW = NC * NS
B_PER_W = BATCH // NW
CHUNK = 128
REM = SEQ - CHUNK
NBUF = 2
GROUPS = B_PER_W // NBUF


@functools.partial(
    pl.kernel,
    out_type=jax.ShapeDtypeStruct((BATCH, SEQ, 128), jnp.float32),
    mesh=plsc.VectorSubcoreMesh(core_axis_name="c", subcore_axis_name="s"),
    scratch_types=[
        pltpu.VMEM((B_PER_W, SEQ), jnp.int32),
        pltpu.VMEM((NBUF, SEQ, 128), jnp.float32),
        pltpu.SemaphoreType.DMA((NBUF,)),
        pltpu.SemaphoreType.DMA((NBUF,)),
    ],
)
def _gather_kernel(table_hbm, idx_hbm, out_hbm, idx_v, rows_v, gsem, wsem):
    wid = lax.axis_index("s") * NC + lax.axis_index("c")
    base = wid * B_PER_W
    pltpu.sync_copy(idx_hbm.at[wid], idx_v)

    def gathers(i, b):
        return (
            pltpu.make_async_copy(
                table_hbm.at[idx_v.at[i, pl.ds(0, CHUNK)]],
                rows_v.at[b, pl.ds(0, CHUNK)], gsem.at[b]),
            pltpu.make_async_copy(
                table_hbm.at[idx_v.at[i, pl.ds(CHUNK, REM)]],
                rows_v.at[b, pl.ds(CHUNK, REM)], gsem.at[b]),
        )

    def writeback(i, b):
        return pltpu.make_async_copy(
            rows_v.at[b], out_hbm.at[base + i], wsem.at[b])

    for b in range(NBUF):
        for c in gathers(b, b):
            c.start()

    def group(g, carry):
        i0 = g * NBUF
        for b in range(NBUF):
            for c in gathers(i0 + b, b):
                c.wait()
            writeback(i0 + b, b).start()
        for b in range(NBUF):
            writeback(i0 + b, b).wait()

            @pl.when(g + 1 < GROUPS)
            def _():
                for c in gathers(i0 + NBUF + b, b):
                    c.start()

        return carry

    lax.fori_loop(0, GROUPS, group, 0)


def kernel(inp_seq, inp_seq_len, embedding_table):
    del inp_seq_len
    table_pad = jnp.pad(embedding_table, ((0, 0), (0, 128 - HIDDEN)))
    idx = inp_seq.astype(jnp.int32).reshape(NW, B_PER_W, SEQ)
    return _gather_kernel(table_pad, idx)[..., :HIDDEN]


# speedup vs baseline: 1.9333x; 1.0095x over previous
import functools

import jax
import jax.numpy as jnp
from jax import lax
from jax.experimental import pallas as pl
from jax.experimental.pallas import tpu as pltpu
from jax.experimental.pallas import tpu_sc as plsc

BATCH = 4096
SEQ = 200
HIDDEN = 64
VOCAB = 1000000
PADW = 128

_info = plsc.get_sparse_core_info()
NC, NS = _info.num_cores, _info.num_subcores
NW = NC * NS
IDX_PER_W = BATCH * SEQ // NW
CHUNK = 128
NCHUNK = IDX_PER_W // CHUNK
NBUF = 5
GROUPS = NCHUNK // NBUF


@functools.partial(
    pl.kernel,
    out_type=jax.ShapeDtypeStruct((NW, IDX_PER_W, PADW), jnp.float32),
    mesh=plsc.VectorSubcoreMesh(core_axis_name="c", subcore_axis_name="s"),
    scratch_types=[
        pltpu.VMEM((IDX_PER_W,), jnp.int32),
        pltpu.VMEM((NBUF, CHUNK, PADW), jnp.float32),
        pltpu.SemaphoreType.DMA((NBUF,)),
        pltpu.SemaphoreType.DMA((NBUF,)),
    ],
)
def _gather_kernel(table_hbm, idx_hbm, out_hbm, idx_v, rows_v, gsem, wsem):
    wid = lax.axis_index("s") * NC + lax.axis_index("c")
    pltpu.sync_copy(idx_hbm.at[wid], idx_v)

    def gather(c, b):
        return pltpu.make_async_copy(
            table_hbm.at[idx_v.at[pl.ds(c * CHUNK, CHUNK)]],
            rows_v.at[b], gsem.at[b])

    def writeback(c, b):
        return pltpu.make_async_copy(
            rows_v.at[b], out_hbm.at[wid, pl.ds(c * CHUNK, CHUNK)], wsem.at[b])

    for b in range(NBUF):
        gather(b, b).start()

    def group(g, carry):
        c0 = g * NBUF
        for b in range(NBUF):
            gather(c0 + b, b).wait()
            writeback(c0 + b, b).start()
        for b in range(NBUF):
            writeback(c0 + b, b).wait()

            @pl.when(g + 1 < GROUPS)
            def _():
                gather(c0 + NBUF + b, b).start()

        return carry

    lax.fori_loop(0, GROUPS, group, 0)


def kernel(inp_seq, inp_seq_len, embedding_table):
    del inp_seq_len
    table_pad = jnp.pad(embedding_table, ((0, 0), (0, PADW - HIDDEN)))
    idx = inp_seq.astype(jnp.int32).reshape(NW, IDX_PER_W)
    out = _gather_kernel(table_pad, idx)
    return out[:, :, :HIDDEN].reshape(BATCH, SEQ, HIDDEN)
